# baseline (speedup 1.0000x reference)
import functools

import jax
import jax.numpy as jnp
from jax import lax
from jax.experimental import pallas as pl
from jax.experimental.pallas import tpu as pltpu
from jax.experimental.pallas import tpu_sc as plsc

VOCAB = 50304
DIM = 768
BATCH = 2
SEQ = 2048
N_TOK = BATCH * SEQ
NC = 2
NS = 16
NW = NC * NS
B_PER_W = N_TOK // NW
CH = 64
NCHUNK = B_PER_W // CH
NBUF = 2

_mesh = plsc.VectorSubcoreMesh(core_axis_name="c", subcore_axis_name="s")


def _make_gather(n_tables, dup):
    n_out = n_tables * (2 if dup else 1)

    @functools.partial(
        pl.kernel,
        out_type=[jax.ShapeDtypeStruct((N_TOK, DIM), jnp.float32)] * n_out,
        mesh=_mesh,
        scratch_types=[
            pltpu.VMEM((B_PER_W,), jnp.int32),
        ] + [pltpu.VMEM((CH, DIM), jnp.float32)] * NBUF
          + [pltpu.SemaphoreType.DMA] * (2 * NBUF),
    )
    def body(idx_hbm, *refs):
        tables = refs[:n_tables]
        outs = refs[n_tables:n_tables + n_out]
        idx_v = refs[n_tables + n_out]
        bufs = refs[n_tables + n_out + 1:n_tables + n_out + 1 + NBUF]
        sems = refs[n_tables + n_out + 1 + NBUF:]
        gsems = sems[:NBUF]
        osems = sems[NBUF:]
        wid = lax.axis_index("s") * NC + lax.axis_index("c")
        base = wid * B_PER_W
        w_per_b = SEQ // B_PER_W
        pltpu.sync_copy(
            idx_hbm.at[wid // w_per_b,
                       pl.ds((wid % w_per_b) * B_PER_W, B_PER_W)], idx_v)
        steps = []
        for t in range(n_tables):
            for c in range(NCHUNK):
                off = c * CH
                first = t == 0 and c == 0
                last = t == n_tables - 1 and c == NCHUNK - 1
                if first or last:
                    steps.append((t, off, CH // 2))
                    steps.append((t, off + CH // 2, CH // 2))
                else:
                    steps.append((t, off, CH))
        nsteps = len(steps)

        def start_gather(s):
            t, off, n = steps[s]
            b = s % NBUF
            dst = bufs[b] if n == CH else bufs[b].at[pl.ds(0, n)]
            return pltpu.async_copy(
                tables[t].at[idx_v.at[pl.ds(off, n)]], dst, gsems[b])

        def start_out(s):
            t, off, n = steps[s]
            b = s % NBUF
            src = bufs[b] if n == CH else bufs[b].at[pl.ds(0, n)]
            ds = pl.ds(base + off, n)
            d = [pltpu.async_copy(src, outs[t].at[ds], osems[b])]
            if dup:
                d.append(pltpu.async_copy(
                    src, outs[n_tables + t].at[ds], osems[b]))
            return d

        g_desc = {0: start_gather(0)}
        o_desc = {}
        for s in range(nsteps):
            if s + 1 < nsteps:
                if s + 1 - NBUF >= 0:
                    for d in o_desc[s + 1 - NBUF]:
                        d.wait()
                g_desc[s + 1] = start_gather(s + 1)
            g_desc[s].wait()
            o_desc[s] = start_out(s)
        for s in range(max(0, nsteps - NBUF), nsteps):
            for d in o_desc[s]:
                d.wait()

    return body


_gather_dup6 = _make_gather(6, dup=True)


def kernel(inputs, W0, W1, W2, W3, W4, W5):
    idx = inputs.astype(jnp.int32)
    outs = _gather_dup6(idx, W0, W1, W2, W3, W4, W5)
    sh = (BATCH, SEQ, DIM)
    prim = list(outs[:6])
    dups = list(outs[6:])
    ve = prim + dups[::-1]
    return tuple(o.reshape(sh) for o in ve)


# speedup vs baseline: 1.0333x; 1.0333x over previous
import functools

import jax
import jax.numpy as jnp
from jax import lax
from jax.experimental import pallas as pl
from jax.experimental.pallas import tpu as pltpu
from jax.experimental.pallas import tpu_sc as plsc

VOCAB = 50304
DIM = 768
BATCH = 2
SEQ = 2048
N_TOK = BATCH * SEQ
NC = 2
NS = 16
NW = NC * NS
B_PER_W = N_TOK // NW
CH = 64
NCHUNK = B_PER_W // CH
NBUF = 2

_mesh = plsc.VectorSubcoreMesh(core_axis_name="c", subcore_axis_name="s")


def _make_gather(n_tables, dup):
    n_out = n_tables * (2 if dup else 1)

    @functools.partial(
        pl.kernel,
        out_type=[jax.ShapeDtypeStruct((N_TOK, DIM), jnp.float32)] * n_out,
        mesh=_mesh,
        scratch_types=[
            pltpu.VMEM((B_PER_W,), jnp.int32),
        ] + [pltpu.VMEM((CH, DIM), jnp.float32)] * NBUF
          + [pltpu.SemaphoreType.DMA] * (2 * NBUF),
    )
    def body(idx_hbm, *refs):
        tables = refs[:n_tables]
        outs = refs[n_tables:n_tables + n_out]
        idx_v = refs[n_tables + n_out]
        bufs = refs[n_tables + n_out + 1:n_tables + n_out + 1 + NBUF]
        sems = refs[n_tables + n_out + 1 + NBUF:]
        gsems = sems[:NBUF]
        osems = sems[NBUF:]
        wid = lax.axis_index("s") * NC + lax.axis_index("c")
        base = wid * B_PER_W
        w_per_b = SEQ // B_PER_W
        pltpu.sync_copy(
            idx_hbm.at[wid // w_per_b,
                       pl.ds((wid % w_per_b) * B_PER_W, B_PER_W)], idx_v)
        steps = [(t, c * CH, CH)
                 for t in range(n_tables) for c in range(NCHUNK)]
        nsteps = len(steps)

        def start_gather(s):
            t, off, n = steps[s]
            b = s % NBUF
            dst = bufs[b] if n == CH else bufs[b].at[pl.ds(0, n)]
            return pltpu.async_copy(
                tables[t].at[idx_v.at[pl.ds(off, n)]], dst, gsems[b])

        def start_out(s):
            t, off, n = steps[s]
            b = s % NBUF
            src = bufs[b] if n == CH else bufs[b].at[pl.ds(0, n)]
            ds = pl.ds(base + off, n)
            d = [pltpu.async_copy(src, outs[t].at[ds], osems[b])]
            if dup:
                d.append(pltpu.async_copy(
                    src, outs[n_tables + t].at[ds], osems[b]))
            return d

        g_desc = {0: start_gather(0)}
        o_desc = {}
        for s in range(nsteps):
            if s + 1 < nsteps:
                if s + 1 - NBUF >= 0:
                    for d in o_desc[s + 1 - NBUF]:
                        d.wait()
                g_desc[s + 1] = start_gather(s + 1)
            g_desc[s].wait()
            o_desc[s] = start_out(s)
        for s in range(max(0, nsteps - NBUF), nsteps):
            for d in o_desc[s]:
                d.wait()

    return body


_gather_dup6 = _make_gather(6, dup=True)


def kernel(inputs, W0, W1, W2, W3, W4, W5):
    idx = inputs.astype(jnp.int32)
    outs = _gather_dup6(idx, W0, W1, W2, W3, W4, W5)
    sh = (BATCH, SEQ, DIM)
    prim = list(outs[:6])
    dups = list(outs[6:])
    ve = prim + dups[::-1]
    return tuple(o.reshape(sh) for o in ve)
